# baseline (speedup 1.0000x reference)
import numpy as np

import jax
import jax.numpy as jnp
from jax.experimental import pallas as pl
from jax.experimental.pallas import tpu as pltpu

_BN_EPS = 1e-5
_L2_EPS = 1e-12


def _l2_normalize(x):
    ss = jnp.sum(x * x, axis=-1, keepdims=True)
    return x * jax.lax.rsqrt(jnp.maximum(ss, _L2_EPS * _L2_EPS))


def _bn_affine(v, g, b):
    mu = jnp.mean(v, axis=0, keepdims=True)
    var = jnp.mean(jnp.square(v - mu), axis=0, keepdims=True)
    return (v - mu) * jax.lax.rsqrt(var + _BN_EPS) * g + b


def _projector_body(x, w1, g1, b1, w2, g2, b2):
    h = jnp.dot(x, w1, preferred_element_type=jnp.float32)
    h = _bn_affine(h, g1, b1)
    h = jnp.maximum(h, 0.0)
    z = jnp.dot(h, w2, preferred_element_type=jnp.float32)
    return _bn_affine(z, g2, b2)


def _pool_matrix(H, W, ph, pw):
    kh, kw = H // ph, W // pw
    py = np.arange(H) // kh
    px = np.arange(W) // kw
    patch_id = (py[:, None] * pw + px[None, :]).reshape(H * W)
    onehot = (patch_id[None, :] == np.arange(ph * pw)[:, None])
    patch_rows = onehot.astype(np.float32) / float(kh * kw)
    global_row = np.full((1, H * W), 1.0 / float(H * W), np.float32)
    return np.concatenate([patch_rows, global_row], axis=0)


def _pool_encode_kernel(pm_ref, xi_ref, xj_ref, wcat_ref, wenc_ref,
                        op_ref, og_ref, ot_ref, *, emb, n_patch):
    pm = pm_ref[...]
    pooled_i = jnp.dot(pm, xi_ref[...], preferred_element_type=jnp.float32)
    fi = jnp.dot(pooled_i, wcat_ref[...], preferred_element_type=jnp.float32)
    op_ref[0, 0] = fi[:n_patch, :emb]
    og_ref[0, 0, 0] = fi[n_patch, :emb]
    ot_ref[0, 0] = fi[n_patch, emb:]
    pooled_j = jnp.dot(pm, xj_ref[...], preferred_element_type=jnp.float32)
    fj = jnp.dot(pooled_j, wenc_ref[...], preferred_element_type=jnp.float32)
    op_ref[1, 0] = fj[:n_patch, :]
    og_ref[1, 0, 0] = fj[n_patch, :]


def _heads_kernel(patch_ref, glob_ref, ht_ref, wp_ref, bp_ref,
                  w1_ref, g1_ref, b1_ref, w2_ref, g2_ref, b2_ref,
                  q1_ref, qg1_ref, qb1_ref, q2_ref, qg2_ref, qb2_ref,
                  zp_ref, zg_ref, xn_ref, zt_ref):
    k = pl.program_id(0)
    w1, g1, b1 = w1_ref[...], g1_ref[...], b1_ref[...]
    w2, g2, b2 = w2_ref[...], g2_ref[...], b2_ref[...]

    xp = _l2_normalize(patch_ref[...])
    zp_ref[...] = _projector_body(xp, w1, g1, b1, w2, g2, b2)

    xg = _l2_normalize(glob_ref[...])
    xn_ref[...] = xg
    zg_ref[...] = _projector_body(xg, w1, g1, b1, w2, g2, b2)

    ht = ht_ref[...]
    hp = jnp.dot(ht, wp_ref[...], preferred_element_type=jnp.float32) + bp_ref[...]
    xt = jnp.where(k == 0, _l2_normalize(ht), hp)
    zt_ref[...] = _projector_body(xt, q1_ref[...], qg1_ref[...], qb1_ref[...],
                                  q2_ref[...], qg2_ref[...], qb2_ref[...])


def kernel(x_i, x_j, w_enc, w_enc_T, w_pred, b_pred,
           proj_w1, proj_g1, proj_b1, proj_w2, proj_g2, proj_b2,
           proj2_w1, proj2_g1, proj2_b1, proj2_w2, proj2_g2, proj2_b2):
    import functools

    B, C, H, W = x_i.shape
    HW = H * W
    ph, pw = 4, 4
    n_patch = ph * pw
    PP = n_patch + 1
    emb = w_enc.shape[1]
    t_dim = w_enc_T.shape[1]
    d_out = proj_w2.shape[1]

    pm = jnp.asarray(_pool_matrix(H, W, ph, pw)).astype(jnp.bfloat16)
    w_cat = jnp.concatenate([w_enc, w_enc_T], axis=1)
    xfi = jnp.transpose(x_i, (0, 2, 3, 1)).reshape(B, HW, C).astype(jnp.bfloat16)
    xfj = jnp.transpose(x_j, (0, 2, 3, 1)).reshape(B, HW, C).astype(jnp.bfloat16)

    pool_kern = functools.partial(_pool_encode_kernel, emb=emb,
                                  n_patch=n_patch)
    h_patch, h_glob4, h_t3 = pl.pallas_call(
        pool_kern,
        out_shape=(
            jax.ShapeDtypeStruct((2, B, n_patch, emb), jnp.float32),
            jax.ShapeDtypeStruct((2, B, 1, emb), jnp.float32),
            jax.ShapeDtypeStruct((B, 1, t_dim), jnp.float32),
        ),
        grid=(B,),
        in_specs=[
            pl.BlockSpec((PP, HW), lambda b: (0, 0)),
            pl.BlockSpec((None, HW, C), lambda b: (b, 0, 0)),
            pl.BlockSpec((None, HW, C), lambda b: (b, 0, 0)),
            pl.BlockSpec((C, emb + t_dim), lambda b: (0, 0)),
            pl.BlockSpec((C, emb), lambda b: (0, 0)),
        ],
        out_specs=(
            pl.BlockSpec((2, 1, n_patch, emb), lambda b: (0, b, 0, 0)),
            pl.BlockSpec((2, 1, 1, emb), lambda b: (0, b, 0, 0)),
            pl.BlockSpec((1, 1, t_dim), lambda b: (b, 0, 0)),
        ),
        compiler_params=pltpu.CompilerParams(dimension_semantics=("parallel",)),
    )(pm, xfi, xfj, w_cat, w_enc)

    patch_stack = h_patch.reshape(2, B * n_patch, emb)
    h_glob = h_glob4.reshape(2, B, emb)
    h_t = h_t3.reshape(B, t_dim)
    NP = B * n_patch

    zp, zg, xn, zt = pl.pallas_call(
        _heads_kernel,
        out_shape=(
            jax.ShapeDtypeStruct((2, NP, d_out), jnp.float32),
            jax.ShapeDtypeStruct((2, B, d_out), jnp.float32),
            jax.ShapeDtypeStruct((2, B, emb), jnp.float32),
            jax.ShapeDtypeStruct((2, B, d_out), jnp.float32),
        ),
        grid=(2,),
        in_specs=[
            pl.BlockSpec((None, NP, emb), lambda k: (k, 0, 0)),
            pl.BlockSpec((None, B, emb), lambda k: (k, 0, 0)),
            pl.BlockSpec((B, t_dim), lambda k: (0, 0)),
            pl.BlockSpec((t_dim, t_dim), lambda k: (0, 0)),
            pl.BlockSpec((1, t_dim), lambda k: (0, 0)),
            pl.BlockSpec((emb, emb), lambda k: (0, 0)),
            pl.BlockSpec((1, emb), lambda k: (0, 0)),
            pl.BlockSpec((1, emb), lambda k: (0, 0)),
            pl.BlockSpec((emb, d_out), lambda k: (0, 0)),
            pl.BlockSpec((1, d_out), lambda k: (0, 0)),
            pl.BlockSpec((1, d_out), lambda k: (0, 0)),
            pl.BlockSpec((t_dim, t_dim), lambda k: (0, 0)),
            pl.BlockSpec((1, t_dim), lambda k: (0, 0)),
            pl.BlockSpec((1, t_dim), lambda k: (0, 0)),
            pl.BlockSpec((t_dim, d_out), lambda k: (0, 0)),
            pl.BlockSpec((1, d_out), lambda k: (0, 0)),
            pl.BlockSpec((1, d_out), lambda k: (0, 0)),
        ],
        out_specs=(
            pl.BlockSpec((None, NP, d_out), lambda k: (k, 0, 0)),
            pl.BlockSpec((None, B, d_out), lambda k: (k, 0, 0)),
            pl.BlockSpec((None, B, emb), lambda k: (k, 0, 0)),
            pl.BlockSpec((None, B, d_out), lambda k: (k, 0, 0)),
        ),
        compiler_params=pltpu.CompilerParams(dimension_semantics=("parallel",)),
    )(patch_stack, h_glob, h_t, w_pred, b_pred,
      proj_w1, proj_g1, proj_b1, proj_w2, proj_g2, proj_b2,
      proj2_w1, proj2_g1, proj2_b1, proj2_w2, proj2_g2, proj2_b2)

    return (zg[0], zg[1], zp[0], zp[1], zt[1], zt[0], h_glob[0], xn[0])


# speedup vs baseline: 1.8721x; 1.8721x over previous
import numpy as np

import jax
import jax.numpy as jnp
from jax.experimental import pallas as pl
from jax.experimental.pallas import tpu as pltpu

_BN_EPS = 1e-5
_L2_EPS = 1e-12


def _l2_normalize(x):
    ss = jnp.sum(x * x, axis=-1, keepdims=True)
    return x * jax.lax.rsqrt(jnp.maximum(ss, _L2_EPS * _L2_EPS))


def _bn_affine(v, g, b):
    mu = jnp.mean(v, axis=0, keepdims=True)
    var = jnp.mean(jnp.square(v - mu), axis=0, keepdims=True)
    return (v - mu) * jax.lax.rsqrt(var + _BN_EPS) * g + b


def _projector_body(x, w1, g1, b1, w2, g2, b2):
    h = jnp.dot(x, w1, preferred_element_type=jnp.float32)
    h = _bn_affine(h, g1, b1)
    h = jnp.maximum(h, 0.0)
    z = jnp.dot(h, w2, preferred_element_type=jnp.float32)
    return _bn_affine(z, g2, b2)


def _pool_matrix(H, W, ph, pw):
    kh, kw = H // ph, W // pw
    py = np.arange(H) // kh
    px = np.arange(W) // kw
    patch_id = (py[:, None] * pw + px[None, :]).reshape(H * W)
    onehot = (patch_id[None, :] == np.arange(ph * pw)[:, None])
    patch_rows = onehot.astype(np.float32) / float(kh * kw)
    global_row = np.full((1, H * W), 1.0 / float(H * W), np.float32)
    return np.concatenate([patch_rows, global_row], axis=0)


def _pool_encode_kernel(pm_ref, xi_ref, xj_ref, wcat_ref, wenc_ref,
                        op_ref, og_ref, ot_ref, *, emb, n_patch):
    pm = pm_ref[...]
    for n in range(xi_ref.shape[0]):
        pooled_i = jnp.dot(pm, xi_ref[n], preferred_element_type=jnp.float32)
        fi = jnp.dot(pooled_i, wcat_ref[...], preferred_element_type=jnp.float32)
        op_ref[0, n] = fi[:n_patch, :emb]
        og_ref[0, n, 0] = fi[n_patch, :emb]
        ot_ref[n, 0] = fi[n_patch, emb:]
        pooled_j = jnp.dot(pm, xj_ref[n], preferred_element_type=jnp.float32)
        fj = jnp.dot(pooled_j, wenc_ref[...], preferred_element_type=jnp.float32)
        op_ref[1, n] = fj[:n_patch, :]
        og_ref[1, n, 0] = fj[n_patch, :]


def _heads_kernel(patch_ref, glob_ref, ht_ref, wp_ref, bp_ref,
                  w1_ref, g1_ref, b1_ref, w2_ref, g2_ref, b2_ref,
                  q1_ref, qg1_ref, qb1_ref, q2_ref, qg2_ref, qb2_ref,
                  zp_ref, zg_ref, xn_ref, zt_ref):
    k = pl.program_id(0)
    w1, g1, b1 = w1_ref[...], g1_ref[...], b1_ref[...]
    w2, g2, b2 = w2_ref[...], g2_ref[...], b2_ref[...]

    xp = _l2_normalize(patch_ref[...])
    zp_ref[...] = _projector_body(xp, w1, g1, b1, w2, g2, b2)

    xg = _l2_normalize(glob_ref[...])
    xn_ref[...] = xg
    zg_ref[...] = _projector_body(xg, w1, g1, b1, w2, g2, b2)

    ht = ht_ref[...]
    hp = jnp.dot(ht, wp_ref[...], preferred_element_type=jnp.float32) + bp_ref[...]
    xt = jnp.where(k == 0, _l2_normalize(ht), hp)
    zt_ref[...] = _projector_body(xt, q1_ref[...], qg1_ref[...], qb1_ref[...],
                                  q2_ref[...], qg2_ref[...], qb2_ref[...])


def kernel(x_i, x_j, w_enc, w_enc_T, w_pred, b_pred,
           proj_w1, proj_g1, proj_b1, proj_w2, proj_g2, proj_b2,
           proj2_w1, proj2_g1, proj2_b1, proj2_w2, proj2_g2, proj2_b2):
    import functools

    B, C, H, W = x_i.shape
    HW = H * W
    ph, pw = 4, 4
    n_patch = ph * pw
    PP = n_patch + 1
    emb = w_enc.shape[1]
    t_dim = w_enc_T.shape[1]
    d_out = proj_w2.shape[1]

    pm = jnp.asarray(_pool_matrix(H, W, ph, pw))
    w_cat = jnp.concatenate([w_enc, w_enc_T], axis=1)
    xfi = jnp.transpose(x_i, (0, 2, 3, 1)).reshape(B, HW, C)
    xfj = jnp.transpose(x_j, (0, 2, 3, 1)).reshape(B, HW, C)

    pool_kern = functools.partial(_pool_encode_kernel, emb=emb,
                                  n_patch=n_patch)
    h_patch, h_glob4, h_t3 = pl.pallas_call(
        pool_kern,
        out_shape=(
            jax.ShapeDtypeStruct((2, B, n_patch, emb), jnp.float32),
            jax.ShapeDtypeStruct((2, B, 1, emb), jnp.float32),
            jax.ShapeDtypeStruct((B, 1, t_dim), jnp.float32),
        ),
        grid=(B // 2,),
        in_specs=[
            pl.BlockSpec((PP, HW), lambda b: (0, 0)),
            pl.BlockSpec((2, HW, C), lambda b: (b, 0, 0)),
            pl.BlockSpec((2, HW, C), lambda b: (b, 0, 0)),
            pl.BlockSpec((C, emb + t_dim), lambda b: (0, 0)),
            pl.BlockSpec((C, emb), lambda b: (0, 0)),
        ],
        out_specs=(
            pl.BlockSpec((2, 2, n_patch, emb), lambda b: (0, b, 0, 0)),
            pl.BlockSpec((2, 2, 1, emb), lambda b: (0, b, 0, 0)),
            pl.BlockSpec((2, 1, t_dim), lambda b: (b, 0, 0)),
        ),
        compiler_params=pltpu.CompilerParams(dimension_semantics=("parallel",)),
    )(pm, xfi, xfj, w_cat, w_enc)

    patch_stack = h_patch.reshape(2, B * n_patch, emb)
    h_glob = h_glob4.reshape(2, B, emb)
    h_t = h_t3.reshape(B, t_dim)
    NP = B * n_patch

    zp, zg, xn, zt = pl.pallas_call(
        _heads_kernel,
        out_shape=(
            jax.ShapeDtypeStruct((2, NP, d_out), jnp.float32),
            jax.ShapeDtypeStruct((2, B, d_out), jnp.float32),
            jax.ShapeDtypeStruct((2, B, emb), jnp.float32),
            jax.ShapeDtypeStruct((2, B, d_out), jnp.float32),
        ),
        grid=(2,),
        in_specs=[
            pl.BlockSpec((None, NP, emb), lambda k: (k, 0, 0)),
            pl.BlockSpec((None, B, emb), lambda k: (k, 0, 0)),
            pl.BlockSpec((B, t_dim), lambda k: (0, 0)),
            pl.BlockSpec((t_dim, t_dim), lambda k: (0, 0)),
            pl.BlockSpec((1, t_dim), lambda k: (0, 0)),
            pl.BlockSpec((emb, emb), lambda k: (0, 0)),
            pl.BlockSpec((1, emb), lambda k: (0, 0)),
            pl.BlockSpec((1, emb), lambda k: (0, 0)),
            pl.BlockSpec((emb, d_out), lambda k: (0, 0)),
            pl.BlockSpec((1, d_out), lambda k: (0, 0)),
            pl.BlockSpec((1, d_out), lambda k: (0, 0)),
            pl.BlockSpec((t_dim, t_dim), lambda k: (0, 0)),
            pl.BlockSpec((1, t_dim), lambda k: (0, 0)),
            pl.BlockSpec((1, t_dim), lambda k: (0, 0)),
            pl.BlockSpec((t_dim, d_out), lambda k: (0, 0)),
            pl.BlockSpec((1, d_out), lambda k: (0, 0)),
            pl.BlockSpec((1, d_out), lambda k: (0, 0)),
        ],
        out_specs=(
            pl.BlockSpec((None, NP, d_out), lambda k: (k, 0, 0)),
            pl.BlockSpec((None, B, d_out), lambda k: (k, 0, 0)),
            pl.BlockSpec((None, B, emb), lambda k: (k, 0, 0)),
            pl.BlockSpec((None, B, d_out), lambda k: (k, 0, 0)),
        ),
        compiler_params=pltpu.CompilerParams(dimension_semantics=("parallel",)),
    )(patch_stack, h_glob, h_t, w_pred, b_pred,
      proj_w1, proj_g1, proj_b1, proj_w2, proj_g2, proj_b2,
      proj2_w1, proj2_g1, proj2_b1, proj2_w2, proj2_g2, proj2_b2)

    return (zg[0], zg[1], zp[0], zp[1], zt[1], zt[0], h_glob[0], xn[0])


# speedup vs baseline: 2.1464x; 1.1465x over previous
import numpy as np

import jax
import jax.numpy as jnp
from jax.experimental import pallas as pl
from jax.experimental.pallas import tpu as pltpu

_BN_EPS = 1e-5
_L2_EPS = 1e-12


def _l2_normalize(x):
    ss = jnp.sum(x * x, axis=-1, keepdims=True)
    return x * jax.lax.rsqrt(jnp.maximum(ss, _L2_EPS * _L2_EPS))


def _bn_affine(v, g, b):
    mu = jnp.mean(v, axis=0, keepdims=True)
    var = jnp.mean(jnp.square(v - mu), axis=0, keepdims=True)
    return (v - mu) * jax.lax.rsqrt(var + _BN_EPS) * g + b


def _projector_body(x, w1, g1, b1, w2, g2, b2):
    h = jnp.dot(x, w1, preferred_element_type=jnp.float32)
    h = _bn_affine(h, g1, b1)
    h = jnp.maximum(h, 0.0)
    z = jnp.dot(h, w2, preferred_element_type=jnp.float32)
    return _bn_affine(z, g2, b2)


def _pool_matrix(H, W, ph, pw):
    kh, kw = H // ph, W // pw
    py = np.arange(H) // kh
    px = np.arange(W) // kw
    patch_id = (py[:, None] * pw + px[None, :]).reshape(H * W)
    onehot = (patch_id[None, :] == np.arange(ph * pw)[:, None])
    patch_rows = onehot.astype(np.float32) / float(kh * kw)
    global_row = np.full((1, H * W), 1.0 / float(H * W), np.float32)
    return np.concatenate([patch_rows, global_row], axis=0)


def _pool_encode_kernel(pm_ref, xi_ref, xj_ref, wcat_ref, wenc_ref,
                        op_ref, og_ref, ot_ref, *, emb, n_patch):
    pm = pm_ref[...]
    for n in range(xi_ref.shape[0]):
        pooled_i = jnp.dot(pm, xi_ref[n], preferred_element_type=jnp.float32)
        fi = jnp.dot(pooled_i, wcat_ref[...], preferred_element_type=jnp.float32)
        op_ref[0, n] = fi[:n_patch, :emb]
        og_ref[0, n, 0] = fi[n_patch, :emb]
        ot_ref[n, 0] = fi[n_patch, emb:]
        pooled_j = jnp.dot(pm, xj_ref[n], preferred_element_type=jnp.float32)
        fj = jnp.dot(pooled_j, wenc_ref[...], preferred_element_type=jnp.float32)
        op_ref[1, n] = fj[:n_patch, :]
        og_ref[1, n, 0] = fj[n_patch, :]


def _heads_kernel(patch_ref, glob_ref, ht_ref, wp_ref, bp_ref,
                  w1_ref, g1_ref, b1_ref, w2_ref, g2_ref, b2_ref,
                  q1_ref, qg1_ref, qb1_ref, q2_ref, qg2_ref, qb2_ref,
                  zp_ref, zg_ref, xn_ref, zt_ref):
    k = pl.program_id(0)
    w1, g1, b1 = w1_ref[...], g1_ref[...], b1_ref[...]
    w2, g2, b2 = w2_ref[...], g2_ref[...], b2_ref[...]

    xp = _l2_normalize(patch_ref[...])
    zp_ref[...] = _projector_body(xp, w1, g1, b1, w2, g2, b2)

    xg = _l2_normalize(glob_ref[...])
    xn_ref[...] = xg
    zg_ref[...] = _projector_body(xg, w1, g1, b1, w2, g2, b2)

    ht = ht_ref[...]
    hp = jnp.dot(ht, wp_ref[...], preferred_element_type=jnp.float32) + bp_ref[...]
    xt = jnp.where(k == 0, _l2_normalize(ht), hp)
    zt_ref[...] = _projector_body(xt, q1_ref[...], qg1_ref[...], qb1_ref[...],
                                  q2_ref[...], qg2_ref[...], qb2_ref[...])


def kernel(x_i, x_j, w_enc, w_enc_T, w_pred, b_pred,
           proj_w1, proj_g1, proj_b1, proj_w2, proj_g2, proj_b2,
           proj2_w1, proj2_g1, proj2_b1, proj2_w2, proj2_g2, proj2_b2):
    import functools

    B, C, H, W = x_i.shape
    HW = H * W
    ph, pw = 4, 4
    n_patch = ph * pw
    PP = n_patch + 1
    emb = w_enc.shape[1]
    t_dim = w_enc_T.shape[1]
    d_out = proj_w2.shape[1]

    pm = jnp.asarray(_pool_matrix(H, W, ph, pw))
    w_cat = jnp.concatenate([w_enc, w_enc_T], axis=1)
    xfi = jnp.transpose(x_i, (0, 2, 3, 1)).reshape(B, HW, C)
    xfj = jnp.transpose(x_j, (0, 2, 3, 1)).reshape(B, HW, C)

    pool_kern = functools.partial(_pool_encode_kernel, emb=emb,
                                  n_patch=n_patch)
    h_patch, h_glob4, h_t3 = pl.pallas_call(
        pool_kern,
        out_shape=(
            jax.ShapeDtypeStruct((2, B, n_patch, emb), jnp.float32),
            jax.ShapeDtypeStruct((2, B, 1, emb), jnp.float32),
            jax.ShapeDtypeStruct((B, 1, t_dim), jnp.float32),
        ),
        grid=(B // 4,),
        in_specs=[
            pl.BlockSpec((PP, HW), lambda b: (0, 0)),
            pl.BlockSpec((4, HW, C), lambda b: (b, 0, 0)),
            pl.BlockSpec((4, HW, C), lambda b: (b, 0, 0)),
            pl.BlockSpec((C, emb + t_dim), lambda b: (0, 0)),
            pl.BlockSpec((C, emb), lambda b: (0, 0)),
        ],
        out_specs=(
            pl.BlockSpec((2, 4, n_patch, emb), lambda b: (0, b, 0, 0)),
            pl.BlockSpec((2, 4, 1, emb), lambda b: (0, b, 0, 0)),
            pl.BlockSpec((4, 1, t_dim), lambda b: (b, 0, 0)),
        ),
        compiler_params=pltpu.CompilerParams(dimension_semantics=("parallel",)),
    )(pm, xfi, xfj, w_cat, w_enc)

    patch_stack = h_patch.reshape(2, B * n_patch, emb)
    h_glob = h_glob4.reshape(2, B, emb)
    h_t = h_t3.reshape(B, t_dim)
    NP = B * n_patch

    zp, zg, xn, zt = pl.pallas_call(
        _heads_kernel,
        out_shape=(
            jax.ShapeDtypeStruct((2, NP, d_out), jnp.float32),
            jax.ShapeDtypeStruct((2, B, d_out), jnp.float32),
            jax.ShapeDtypeStruct((2, B, emb), jnp.float32),
            jax.ShapeDtypeStruct((2, B, d_out), jnp.float32),
        ),
        grid=(2,),
        in_specs=[
            pl.BlockSpec((None, NP, emb), lambda k: (k, 0, 0)),
            pl.BlockSpec((None, B, emb), lambda k: (k, 0, 0)),
            pl.BlockSpec((B, t_dim), lambda k: (0, 0)),
            pl.BlockSpec((t_dim, t_dim), lambda k: (0, 0)),
            pl.BlockSpec((1, t_dim), lambda k: (0, 0)),
            pl.BlockSpec((emb, emb), lambda k: (0, 0)),
            pl.BlockSpec((1, emb), lambda k: (0, 0)),
            pl.BlockSpec((1, emb), lambda k: (0, 0)),
            pl.BlockSpec((emb, d_out), lambda k: (0, 0)),
            pl.BlockSpec((1, d_out), lambda k: (0, 0)),
            pl.BlockSpec((1, d_out), lambda k: (0, 0)),
            pl.BlockSpec((t_dim, t_dim), lambda k: (0, 0)),
            pl.BlockSpec((1, t_dim), lambda k: (0, 0)),
            pl.BlockSpec((1, t_dim), lambda k: (0, 0)),
            pl.BlockSpec((t_dim, d_out), lambda k: (0, 0)),
            pl.BlockSpec((1, d_out), lambda k: (0, 0)),
            pl.BlockSpec((1, d_out), lambda k: (0, 0)),
        ],
        out_specs=(
            pl.BlockSpec((None, NP, d_out), lambda k: (k, 0, 0)),
            pl.BlockSpec((None, B, d_out), lambda k: (k, 0, 0)),
            pl.BlockSpec((None, B, emb), lambda k: (k, 0, 0)),
            pl.BlockSpec((None, B, d_out), lambda k: (k, 0, 0)),
        ),
        compiler_params=pltpu.CompilerParams(dimension_semantics=("parallel",)),
    )(patch_stack, h_glob, h_t, w_pred, b_pred,
      proj_w1, proj_g1, proj_b1, proj_w2, proj_g2, proj_b2,
      proj2_w1, proj2_g1, proj2_b1, proj2_w2, proj2_g2, proj2_b2)

    return (zg[0], zg[1], zp[0], zp[1], zt[1], zt[0], h_glob[0], xn[0])
